# baseline (device time: 19893 ns/iter reference)
import jax
import jax.numpy as jnp
from jax import lax
from jax.experimental import pallas as pl
from jax.experimental.pallas import tpu as pltpu

N_DEV = 16


def kernel(Q, K, V):
    b, q, h, d = Q.shape
    kk = K.shape[1]
    scale = d ** -0.5
    Qr = Q.reshape(b, h, d)
    K2 = jnp.transpose(K, (0, 2, 3, 1))
    V2 = jnp.transpose(V, (0, 2, 3, 1))

    def body(q_ref, k_hbm, v_hbm, out_ref, k_ref, v_ref, comm_ref,
             send_sems, recv_sems, ready_sems, load_sems):
        me = lax.axis_index("i")

        barrier_sem = pltpu.get_barrier_semaphore()
        pl.semaphore_signal(barrier_sem, 1)
        pl.semaphore_wait(barrier_sem, 1)

        for off in range(1, N_DEV):
            tgt = lax.rem(me + off, N_DEV)
            pl.semaphore_signal(
                ready_sems.at[me], inc=1,
                device_id=(tgt,), device_id_type=pl.DeviceIdType.MESH,
            )

        bh = b // 2
        k0 = pltpu.make_async_copy(k_hbm.at[0:bh], k_ref.at[0:bh], load_sems.at[0])
        k1 = pltpu.make_async_copy(k_hbm.at[bh:b], k_ref.at[bh:b], load_sems.at[1])
        v0 = pltpu.make_async_copy(v_hbm.at[0:bh], v_ref.at[0:bh], load_sems.at[2])
        v1 = pltpu.make_async_copy(v_hbm.at[bh:b], v_ref.at[bh:b], load_sems.at[3])
        k0.start()
        k1.start()
        v0.start()
        v1.start()
        qv = q_ref[...] * scale

        def partial(lo, hi, kdma, vdma):
            kdma.wait()
            s = jnp.sum(k_ref[lo:hi] * qv[lo:hi, :, :, None], axis=2)
            p = jnp.exp(s)
            l_c = jnp.sum(p, axis=-1)
            vdma.wait()
            o_c = jnp.sum(p[:, :, None, :] * v_ref[lo:hi], axis=-1)
            return o_c, l_c

        o_0, l_0 = partial(0, bh, k0, v0)
        o_1, l_1 = partial(bh, b, k1, v1)

        slot = comm_ref.at[me]
        slot[0:bh, :, :] = o_0.astype(jnp.bfloat16)
        slot[bh:b, :, :] = o_1.astype(jnp.bfloat16)
        slot[b, 0:bh, 0:h] = l_0.astype(jnp.bfloat16)
        slot[b, bh:b, 0:h] = l_1.astype(jnp.bfloat16)

        rdmas = []
        for off in range(1, N_DEV):
            tgt = lax.rem(me + off, N_DEV)
            pl.semaphore_wait(ready_sems.at[tgt], 1)
            rdma = pltpu.make_async_remote_copy(
                src_ref=comm_ref.at[me],
                dst_ref=comm_ref.at[me],
                send_sem=send_sems.at[off - 1],
                recv_sem=recv_sems.at[me],
                device_id=(tgt,),
                device_id_type=pl.DeviceIdType.MESH,
            )
            rdma.start()
            rdmas.append(rdma)

        acc = comm_ref.at[me][...].astype(jnp.float32)
        for kth in range(1, N_DEV):
            src = lax.rem(me - kth + N_DEV, N_DEV)
            recv = pltpu.make_async_remote_copy(
                src_ref=comm_ref.at[me],
                dst_ref=comm_ref.at[src],
                send_sem=send_sems.at[kth - 1],
                recv_sem=recv_sems.at[src],
                device_id=(me,),
                device_id_type=pl.DeviceIdType.MESH,
            )
            recv.wait_recv()
            acc = acc + comm_ref.at[src][...].astype(jnp.float32)

        out_ref[...] = acc[0:b, :, :] / acc[b, :, 0:h][:, :, None]

        for rdma in rdmas:
            rdma.wait_send()

    out = pl.pallas_call(
        body,
        out_shape=jax.ShapeDtypeStruct((b, h, d), jnp.float32),
        in_specs=[
            pl.BlockSpec(memory_space=pltpu.VMEM),
            pl.BlockSpec(memory_space=pltpu.MemorySpace.HBM),
            pl.BlockSpec(memory_space=pltpu.MemorySpace.HBM),
        ],
        out_specs=pl.BlockSpec(memory_space=pltpu.VMEM),
        scratch_shapes=[
            pltpu.VMEM((b, h, d, kk), jnp.float32),
            pltpu.VMEM((b, h, d, kk), jnp.float32),
            pltpu.VMEM((N_DEV, b + 1, h, d), jnp.bfloat16),
            pltpu.SemaphoreType.DMA((N_DEV - 1,)),
            pltpu.SemaphoreType.DMA((N_DEV,)),
            pltpu.SemaphoreType.REGULAR((N_DEV,)),
            pltpu.SemaphoreType.DMA((4,)),
        ],
        compiler_params=pltpu.CompilerParams(collective_id=0),
    )(Qr, K2, V2)
    return out.reshape(b, 1, h, d)


# device time: 18456 ns/iter; 1.0779x vs baseline; 1.0779x over previous
import jax
import jax.numpy as jnp
from jax import lax
from jax.experimental import pallas as pl
from jax.experimental.pallas import tpu as pltpu

N_DEV = 16


def kernel(Q, K, V):
    b, q, h, d = Q.shape
    kk = K.shape[1]
    scale = d ** -0.5
    Qr = Q.reshape(b, h, d)
    K2 = jnp.transpose(K, (0, 2, 3, 1))
    V2 = jnp.transpose(V, (0, 2, 3, 1))

    def body(q_ref, k_ref, v_ref, out_ref, comm_ref,
             send_sems, recv_sems, ready_sems):
        me = lax.axis_index("i")

        barrier_sem = pltpu.get_barrier_semaphore()
        pl.semaphore_signal(barrier_sem, 1)
        pl.semaphore_wait(barrier_sem, 1)

        for off in range(1, N_DEV):
            tgt = lax.rem(me + off, N_DEV)
            pl.semaphore_signal(
                ready_sems.at[me], inc=1,
                device_id=(tgt,), device_id_type=pl.DeviceIdType.MESH,
            )

        qv = q_ref[...] * scale
        s = jnp.sum(k_ref[...] * qv[..., None], axis=2)
        p = jnp.exp(s)
        l_loc = jnp.sum(p, axis=-1)
        o_loc = jnp.sum(p[:, :, None, :] * v_ref[...], axis=-1)

        slot = comm_ref.at[me]
        slot[0:b, :, :] = o_loc.astype(jnp.bfloat16)
        slot[b, :, 0:h] = l_loc.astype(jnp.bfloat16)

        rdmas = []
        for off in range(1, N_DEV):
            tgt = lax.rem(me + off, N_DEV)
            pl.semaphore_wait(ready_sems.at[tgt], 1)
            rdma = pltpu.make_async_remote_copy(
                src_ref=comm_ref.at[me],
                dst_ref=comm_ref.at[me],
                send_sem=send_sems.at[off - 1],
                recv_sem=recv_sems.at[me],
                device_id=(tgt,),
                device_id_type=pl.DeviceIdType.MESH,
            )
            rdma.start()
            rdmas.append(rdma)

        acc = comm_ref.at[me][...].astype(jnp.float32)
        for kth in range(1, N_DEV):
            src = lax.rem(me - kth + N_DEV, N_DEV)
            recv = pltpu.make_async_remote_copy(
                src_ref=comm_ref.at[me],
                dst_ref=comm_ref.at[src],
                send_sem=send_sems.at[kth - 1],
                recv_sem=recv_sems.at[src],
                device_id=(me,),
                device_id_type=pl.DeviceIdType.MESH,
            )
            recv.wait_recv()
            acc = acc + comm_ref.at[src][...].astype(jnp.float32)

        out_ref[...] = acc[0:b, :, :] / acc[b, :, 0:h][:, :, None]

        for rdma in rdmas:
            rdma.wait_send()

    out = pl.pallas_call(
        body,
        out_shape=jax.ShapeDtypeStruct((b, h, d), jnp.float32),
        in_specs=[
            pl.BlockSpec(memory_space=pltpu.VMEM),
            pl.BlockSpec(memory_space=pltpu.VMEM),
            pl.BlockSpec(memory_space=pltpu.VMEM),
        ],
        out_specs=pl.BlockSpec(memory_space=pltpu.VMEM),
        scratch_shapes=[
            pltpu.VMEM((N_DEV, b + 1, h, d), jnp.bfloat16),
            pltpu.SemaphoreType.DMA((N_DEV - 1,)),
            pltpu.SemaphoreType.DMA((N_DEV,)),
            pltpu.SemaphoreType.REGULAR((N_DEV,)),
        ],
        compiler_params=pltpu.CompilerParams(collective_id=0),
    )(Qr, K2, V2)
    return out.reshape(b, 1, h, d)


# device time: 18433 ns/iter; 1.0792x vs baseline; 1.0012x over previous
import jax
import jax.numpy as jnp
from jax import lax
from jax.experimental import pallas as pl
from jax.experimental.pallas import tpu as pltpu

N_DEV = 16


def kernel(Q, K, V):
    b, q, h, d = Q.shape
    kk = K.shape[1]
    scale = d ** -0.5
    Qr = Q.reshape(b, h, d)
    K2 = jnp.transpose(K, (0, 2, 3, 1))
    V2 = jnp.transpose(V, (0, 2, 3, 1))

    def body(q_ref, k_ref, v_ref, out_ref, comm_ref,
             send_sems, recv_sems, ready_sems):
        me = lax.axis_index("i")

        barrier_sem = pltpu.get_barrier_semaphore()
        pl.semaphore_signal(barrier_sem, 1)
        pl.semaphore_wait(barrier_sem, 1)

        for off in range(1, N_DEV):
            tgt = lax.rem(me + off, N_DEV)
            pl.semaphore_signal(
                ready_sems.at[me], inc=1,
                device_id=(tgt,), device_id_type=pl.DeviceIdType.MESH,
            )

        qv = q_ref[...] * scale
        s = jnp.sum(k_ref[...] * qv[..., None], axis=2)
        p = jnp.exp(s)
        l_loc = jnp.sum(p, axis=-1)
        o_loc = jnp.sum(p[:, :, None, :] * v_ref[...], axis=-1)

        slot = comm_ref.at[me]
        slot[0:b, :, :] = o_loc.astype(jnp.bfloat16)
        slot[b, :, 0:h] = l_loc.astype(jnp.bfloat16)

        rdmas = []
        for off in range(1, N_DEV):
            tgt = lax.rem(me + off, N_DEV)
            pl.semaphore_wait(ready_sems.at[tgt], 1)
            rdma = pltpu.make_async_remote_copy(
                src_ref=comm_ref.at[me],
                dst_ref=comm_ref.at[me],
                send_sem=send_sems.at[off - 1],
                recv_sem=recv_sems.at[me],
                device_id=(tgt,),
                device_id_type=pl.DeviceIdType.MESH,
            )
            rdma.start()
            rdmas.append(rdma)

        acc = comm_ref.at[me][...].astype(jnp.float32)
        for kth in range(1, N_DEV):
            src = lax.rem(me - kth + N_DEV, N_DEV)
            recv = pltpu.make_async_remote_copy(
                src_ref=comm_ref.at[me],
                dst_ref=comm_ref.at[src],
                send_sem=send_sems.at[kth - 1],
                recv_sem=recv_sems.at[src],
                device_id=(me,),
                device_id_type=pl.DeviceIdType.MESH,
            )
            recv.wait_recv()
            acc = acc + comm_ref.at[src][...].astype(jnp.float32)

        o_sum = acc[0:b, :, :] / acc[b, :, 0:h][:, :, None]
        out_ref[...] = o_sum[:, None, :, :]

        for rdma in rdmas:
            rdma.wait_send()

    out = pl.pallas_call(
        body,
        out_shape=jax.ShapeDtypeStruct((b, 1, h, d), jnp.float32),
        in_specs=[
            pl.BlockSpec(memory_space=pltpu.VMEM),
            pl.BlockSpec(memory_space=pltpu.VMEM),
            pl.BlockSpec(memory_space=pltpu.VMEM),
        ],
        out_specs=pl.BlockSpec(memory_space=pltpu.VMEM),
        scratch_shapes=[
            pltpu.VMEM((N_DEV, b + 1, h, d), jnp.bfloat16),
            pltpu.SemaphoreType.DMA((N_DEV - 1,)),
            pltpu.SemaphoreType.DMA((N_DEV,)),
            pltpu.SemaphoreType.REGULAR((N_DEV,)),
        ],
        compiler_params=pltpu.CompilerParams(collective_id=0),
    )(Qr, K2, V2)
    return out
